# baseline (speedup 1.0000x reference)
import functools

import jax
import jax.numpy as jnp
from jax.experimental import pallas as pl
from jax.experimental.pallas import tpu as pltpu


def _make_kernel(n_chunks: int, chunk: int):
    def _kernel(w_hbm, x_ref, o_ref, wv_ref, stage_ref, xb_ref, sem):
        def _start(c, slot):
            pltpu.make_async_copy(
                w_hbm.at[pl.ds(c * chunk, chunk)],
                stage_ref.at[slot],
                sem.at[slot],
            ).start()

        @pl.when(pl.program_id(0) == 0)
        def _build_wv_and_compute():
            _start(0, 0)
            xb_ref[...] = x_ref[...].astype(jnp.bfloat16)

            def _body(c, carry):
                cur = jax.lax.rem(c, 2)
                nxt = jax.lax.rem(c + 1, 2)

                @pl.when(c + 1 < n_chunks)
                def _():
                    _start(c + 1, nxt)

                pltpu.make_async_copy(
                    stage_ref.at[cur], stage_ref.at[cur], sem.at[cur]
                ).wait()
                wb_c = stage_ref[cur].astype(jnp.bfloat16)
                wv_ref[pl.ds(c * chunk, chunk), :] = wb_c
                o_ref[:, pl.ds(c * chunk, chunk)] = jax.lax.dot_general(
                    xb_ref[...],
                    wb_c,
                    dimension_numbers=(((1,), (1,)), ((), ())),
                    preferred_element_type=jnp.float32,
                )
                return carry

            jax.lax.fori_loop(0, n_chunks, _body, (), unroll=False)

        @pl.when(pl.program_id(0) > 0)
        def _main():
            o_ref[...] = jax.lax.dot_general(
                x_ref[...].astype(jnp.bfloat16),
                wv_ref[...],
                dimension_numbers=(((1,), (1,)), ((), ())),
                preferred_element_type=jnp.float32,
            )

    return _kernel


def _round_up(v: int, m: int) -> int:
    return -(-v // m) * m


@functools.partial(jax.jit, static_argnames=("bm", "chunk"))
def _linear_no_bias(x, weight, *, bm=256, chunk=128):
    B, K = x.shape
    N, K2 = weight.shape
    assert K == K2, "in_features mismatch"

    bm = min(bm, _round_up(B, 16))
    Bp, Np, Kp = _round_up(B, bm), _round_up(N, 128), _round_up(K, 128)
    if Bp != B or Kp != K:
        x = jnp.pad(x, ((0, Bp - B), (0, Kp - K)))
    if Np != N or Kp != K:
        weight = jnp.pad(weight, ((0, Np - N), (0, Kp - K)))
    chunk = min(chunk, Np)
    n_chunks = -(-Np // chunk)
    assert Np % chunk == 0, "N must divide into prologue chunks"

    out = pl.pallas_call(
        _make_kernel(n_chunks, chunk),
        out_shape=jax.ShapeDtypeStruct((Bp, Np), jnp.float32),
        grid=(Bp // bm,),
        in_specs=[
            pl.BlockSpec(memory_space=pl.ANY),
            pl.BlockSpec((bm, Kp), lambda i: (i, 0)),
        ],
        out_specs=pl.BlockSpec((bm, Np), lambda i: (i, 0)),
        scratch_shapes=[
            pltpu.VMEM((Np, Kp), jnp.bfloat16),
            pltpu.VMEM((2, chunk, Kp), jnp.float32),
            pltpu.VMEM((bm, Kp), jnp.bfloat16),
            pltpu.SemaphoreType.DMA((2,)),
        ],
        compiler_params=pltpu.CompilerParams(
            dimension_semantics=("arbitrary",),
        ),
        cost_estimate=pl.CostEstimate(
            flops=2 * B * N * K,
            transcendentals=0,
            bytes_accessed=B * K * 4 + K * N * 4 + B * N * 4,
        ),
    )(weight, x)

    if Bp != B or Np != N:
        out = out[:B, :N]
    return out


def kernel(x, weight):
    return _linear_no_bias(x, weight)


# speedup vs baseline: 1.0325x; 1.0325x over previous
import functools

import jax
import jax.numpy as jnp
from jax.experimental import pallas as pl
from jax.experimental.pallas import tpu as pltpu


def _make_kernel(n_chunks: int, chunk: int):
    def _kernel(w_hbm, x_ref, o_ref, wv_ref, stage_ref, sem):
        @pl.when(pl.program_id(0) == 0)
        def _build_wv():
            def _start(c, slot):
                pltpu.make_async_copy(
                    w_hbm.at[pl.ds(c * chunk, chunk)],
                    stage_ref.at[slot],
                    sem.at[slot],
                ).start()

            _start(0, 0)

            def _body(c, carry):
                cur = jax.lax.rem(c, 2)
                nxt = jax.lax.rem(c + 1, 2)

                @pl.when(c + 1 < n_chunks)
                def _():
                    _start(c + 1, nxt)

                pltpu.make_async_copy(
                    stage_ref.at[cur], stage_ref.at[cur], sem.at[cur]
                ).wait()
                wv_ref[pl.ds(c * chunk, chunk), :] = stage_ref[cur].astype(
                    jnp.bfloat16
                )
                return carry

            jax.lax.fori_loop(0, n_chunks, _body, (), unroll=False)

        o_ref[...] = jax.lax.dot_general(
            x_ref[...].astype(jnp.bfloat16),
            wv_ref[...],
            dimension_numbers=(((1,), (1,)), ((), ())),
            preferred_element_type=jnp.float32,
        )

    return _kernel


def _round_up(v: int, m: int) -> int:
    return -(-v // m) * m


@functools.partial(jax.jit, static_argnames=("bm", "chunk"))
def _linear_no_bias(x, weight, *, bm=256, chunk=256):
    B, K = x.shape
    N, K2 = weight.shape
    assert K == K2, "in_features mismatch"

    bm = min(bm, _round_up(B, 16))
    Bp, Np, Kp = _round_up(B, bm), _round_up(N, 128), _round_up(K, 128)
    if Bp != B or Kp != K:
        x = jnp.pad(x, ((0, Bp - B), (0, Kp - K)))
    if Np != N or Kp != K:
        weight = jnp.pad(weight, ((0, Np - N), (0, Kp - K)))
    chunk = min(chunk, Np)
    n_chunks = -(-Np // chunk)
    assert Np % chunk == 0, "N must divide into prologue chunks"

    out = pl.pallas_call(
        _make_kernel(n_chunks, chunk),
        out_shape=jax.ShapeDtypeStruct((Bp, Np), jnp.float32),
        grid=(Bp // bm,),
        in_specs=[
            pl.BlockSpec(memory_space=pl.ANY),
            pl.BlockSpec((bm, Kp), lambda i: (i, 0)),
        ],
        out_specs=pl.BlockSpec((bm, Np), lambda i: (i, 0)),
        scratch_shapes=[
            pltpu.VMEM((Np, Kp), jnp.bfloat16),
            pltpu.VMEM((2, chunk, Kp), jnp.float32),
            pltpu.SemaphoreType.DMA((2,)),
        ],
        compiler_params=pltpu.CompilerParams(
            dimension_semantics=("arbitrary",),
        ),
        cost_estimate=pl.CostEstimate(
            flops=2 * B * N * K,
            transcendentals=0,
            bytes_accessed=B * K * 4 + K * N * 4 + B * N * 4,
        ),
    )(weight, x)

    if Bp != B or Np != N:
        out = out[:B, :N]
    return out


def kernel(x, weight):
    return _linear_no_bias(x, weight)
